# baseline (speedup 1.0000x reference)
import functools

import jax
import jax.numpy as jnp
from jax import lax
from jax.experimental import pallas as pl
from jax.experimental.pallas import tpu as pltpu
from jax.experimental.pallas import tpu_sc as plsc

N = 10000
D = 128
E = 320000
BN_EPS = 1e-5

NC = 2
NS = 16
HD = D // NC
CH = 125
NCHUNK = 160
SR = 128
TAB_ROWS = 10240
ROWS_PER_TILE = TAB_ROWS // NS

BC = 16
NBLK = NCHUNK // BC
PAIRS = BC // 2

_mesh = plsc.VectorSubcoreMesh(core_axis_name="c", subcore_axis_name="s")


@functools.partial(
    pl.kernel,
    out_type=jax.ShapeDtypeStruct((TAB_ROWS, D), jnp.float32),
    mesh=_mesh,
    scratch_types=[
        pltpu.VMEM_SHARED((TAB_ROWS, HD), jnp.float32),
        pltpu.VMEM_SHARED((TAB_ROWS, HD), jnp.float32),
        pltpu.VMEM((BC, CH), jnp.int32),
        pltpu.VMEM((BC, CH), jnp.int32),
        pltpu.VMEM((BC, CH), jnp.int32),
        pltpu.VMEM((BC, CH), jnp.int32),
        pltpu.VMEM((SR, HD), jnp.float32),
        pltpu.VMEM((SR, HD), jnp.float32),
        pltpu.SemaphoreType.DMA,
        pltpu.SemaphoreType.DMA,
        pltpu.SemaphoreType.DMA,
        pltpu.SemaphoreType.DMA,
    ],
    compiler_params=pltpu.CompilerParams(use_tc_tiling_on_sc=False),
)
def _sc_segment_sum(x_hbm, edge_hbm, out_hbm,
                    tab, acc, rowA, rowB, colA, colB, gbuf0, gbuf1,
                    gsem0, gsem1, isemA, isemB):
    cid = lax.axis_index("c")
    sid = lax.axis_index("s")
    r0 = sid * ROWS_PER_TILE
    c0col = cid * HD

    for j in range(ROWS_PER_TILE // SR):
        base = r0 + j * SR
        sl = pl.ds(base, SR)

        @pl.when(base + SR <= N)
        def _(sl=sl, base=base):
            pltpu.sync_copy(x_hbm.at[sl, pl.ds(c0col, HD)], tab.at[sl])
            pltpu.sync_copy(x_hbm.at[sl, pl.ds(c0col, HD)], acc.at[sl])

    @pl.when(sid == NS - 1)
    def _():
        tail = N - (N // SR) * SR
        tsl = pl.ds(N - tail, tail)
        pltpu.sync_copy(x_hbm.at[tsl, pl.ds(c0col, HD)], tab.at[tsl])
        pltpu.sync_copy(x_hbm.at[tsl, pl.ds(c0col, HD)], acc.at[tsl])

    plsc.subcore_barrier()

    rows = (rowA, rowB)
    cols = (colA, colB)
    isems = (isemA, isemB)

    pltpu.sync_copy(edge_hbm.at[0, sid, pl.ds(0, BC)], rowA)
    pltpu.sync_copy(edge_hbm.at[1, sid, pl.ds(0, BC)], colA)
    pltpu.async_copy(tab.at[colA.at[0]], gbuf0.at[pl.ds(0, CH)], gsem0)

    for b in range(NBLK):
        ib = b % 2
        rv = rows[ib]
        cv = cols[ib]
        stage = None
        if b + 1 < NBLK:
            ibn = (b + 1) % 2
            stage = (
                pltpu.async_copy(edge_hbm.at[0, sid, pl.ds((b + 1) * BC, BC)],
                                 rows[ibn], isems[ibn]),
                pltpu.async_copy(edge_hbm.at[1, sid, pl.ds((b + 1) * BC, BC)],
                                 cols[ibn], isems[ibn]),
            )

        def pair_body(p, carry, rv=rv, cv=cv):
            c0 = 2 * p
            c1 = c0 + 1
            pltpu.async_copy(tab.at[cv.at[c1]], gbuf1.at[pl.ds(0, CH)], gsem1)
            pltpu.make_async_copy(tab.at[cv.at[c0]], gbuf0.at[pl.ds(0, CH)], gsem0).wait()
            pltpu.sync_copy(gbuf0.at[pl.ds(0, CH)], acc.at[rv.at[c0]], add=True)

            @pl.when(p != PAIRS - 1)
            def _():
                pltpu.async_copy(tab.at[cv.at[c0 + 2]], gbuf0.at[pl.ds(0, CH)], gsem0)

            pltpu.make_async_copy(tab.at[cv.at[c1]], gbuf1.at[pl.ds(0, CH)], gsem1).wait()
            pltpu.sync_copy(gbuf1.at[pl.ds(0, CH)], acc.at[rv.at[c1]], add=True)
            return carry

        lax.fori_loop(0, PAIRS, pair_body, 0)

        if b + 1 < NBLK:
            stage[0].wait()
            stage[1].wait()
            pltpu.async_copy(tab.at[cols[(b + 1) % 2].at[0]], gbuf0.at[pl.ds(0, CH)], gsem0)

    plsc.subcore_barrier()

    for j in range(ROWS_PER_TILE // SR):
        sl = pl.ds(r0 + j * SR, SR)
        pltpu.sync_copy(acc.at[sl], out_hbm.at[sl, pl.ds(c0col, HD)])


def _tc_body(x_ref, p_ref, wt_ref, b_ref, g_ref, bt_ref, eps_ref, o_ref):
    xagg = p_ref[:N, :]
    h = eps_ref[0] * x_ref[...] + xagg
    y = jnp.dot(h, wt_ref[...], preferred_element_type=jnp.float32)
    y = y + b_ref[...]
    mu = jnp.mean(y, axis=0, keepdims=True)
    var = jnp.mean((y - mu) * (y - mu), axis=0, keepdims=True)
    yn = g_ref[...] * (y - mu) / jnp.sqrt(var + BN_EPS) + bt_ref[...]
    o_ref[...] = jnp.where(yn > 0.0, yn, jnp.exp(yn) - 1.0)


_tc_layer = pl.pallas_call(
    _tc_body,
    out_shape=jax.ShapeDtypeStruct((N, D), jnp.float32),
    in_specs=[
        pl.BlockSpec(memory_space=pltpu.VMEM),
        pl.BlockSpec(memory_space=pltpu.VMEM),
        pl.BlockSpec(memory_space=pltpu.VMEM),
        pl.BlockSpec(memory_space=pltpu.VMEM),
        pl.BlockSpec(memory_space=pltpu.VMEM),
        pl.BlockSpec(memory_space=pltpu.VMEM),
        pl.BlockSpec(memory_space=pltpu.SMEM),
    ],
    out_specs=pl.BlockSpec(memory_space=pltpu.VMEM),
)


def kernel(x, edge_index, W1, b1, eps1, g1, bt1, W2, b2, eps2, g2, bt2):
    edges = edge_index.reshape(2, NS, NCHUNK, CH)

    p1 = _sc_segment_sum(x, edges)
    h1 = _tc_layer(x, p1, W1.T, b1[None, :], g1[None, :], bt1[None, :],
                   eps1.reshape(1))
    p2 = _sc_segment_sum(h1, edges)
    h2 = _tc_layer(h1, p2, W2.T, b2[None, :], g2[None, :], bt2[None, :],
                   eps2.reshape(1))
    return h2


# speedup vs baseline: 1.0623x; 1.0623x over previous
import functools

import jax
import jax.numpy as jnp
from jax import lax
from jax.experimental import pallas as pl
from jax.experimental.pallas import tpu as pltpu
from jax.experimental.pallas import tpu_sc as plsc

N = 10000
D = 128
E = 320000
BN_EPS = 1e-5

NC = 2
NS = 16
HD = D // NC
CH = 125
NCHUNK = 160
SR = 128
TAB_ROWS = 10240
ROWS_PER_TILE = TAB_ROWS // NS

BC = 32
NBLK = NCHUNK // BC
PAIRS = BC // 2

_mesh = plsc.VectorSubcoreMesh(core_axis_name="c", subcore_axis_name="s")


@functools.partial(
    pl.kernel,
    out_type=jax.ShapeDtypeStruct((TAB_ROWS, D), jnp.float32),
    mesh=_mesh,
    scratch_types=[
        pltpu.VMEM_SHARED((TAB_ROWS, HD), jnp.float32),
        pltpu.VMEM_SHARED((TAB_ROWS, HD), jnp.float32),
        pltpu.VMEM((BC, CH), jnp.int32),
        pltpu.VMEM((BC, CH), jnp.int32),
        pltpu.VMEM((BC, CH), jnp.int32),
        pltpu.VMEM((BC, CH), jnp.int32),
        pltpu.VMEM((SR, HD), jnp.float32),
        pltpu.VMEM((SR, HD), jnp.float32),
        pltpu.SemaphoreType.DMA,
        pltpu.SemaphoreType.DMA,
        pltpu.SemaphoreType.DMA,
        pltpu.SemaphoreType.DMA,
    ],
    compiler_params=pltpu.CompilerParams(use_tc_tiling_on_sc=False),
)
def _sc_segment_sum(x_hbm, edge_hbm, out_hbm,
                    tab, acc, rowA, rowB, colA, colB, gbuf0, gbuf1,
                    gsem0, gsem1, isemA, isemB):
    cid = lax.axis_index("c")
    sid = lax.axis_index("s")
    r0 = sid * ROWS_PER_TILE
    c0col = cid * HD

    for j in range(ROWS_PER_TILE // SR):
        base = r0 + j * SR
        sl = pl.ds(base, SR)

        @pl.when(base + SR <= N)
        def _(sl=sl, base=base):
            pltpu.sync_copy(x_hbm.at[sl, pl.ds(c0col, HD)], gbuf0)
            pltpu.sync_copy(gbuf0, tab.at[sl])
        pltpu.sync_copy(gbuf0, acc.at[sl])

    @pl.when(sid == NS - 1)
    def _():
        tail = N - (N // SR) * SR
        tsl = pl.ds(N - tail, tail)
        pltpu.sync_copy(x_hbm.at[tsl, pl.ds(c0col, HD)],
                        gbuf0.at[pl.ds(0, tail)])
        pltpu.sync_copy(gbuf0.at[pl.ds(0, tail)], tab.at[tsl])
        pltpu.sync_copy(gbuf0.at[pl.ds(0, tail)], acc.at[tsl])

    plsc.subcore_barrier()

    rows = (rowA, rowB)
    cols = (colA, colB)
    isems = (isemA, isemB)

    pltpu.sync_copy(edge_hbm.at[0, sid, pl.ds(0, BC)], rowA)
    pltpu.sync_copy(edge_hbm.at[1, sid, pl.ds(0, BC)], colA)
    pltpu.async_copy(tab.at[colA.at[0]], gbuf0.at[pl.ds(0, CH)], gsem0)

    for b in range(NBLK):
        ib = b % 2
        rv = rows[ib]
        cv = cols[ib]
        stage = None
        if b + 1 < NBLK:
            ibn = (b + 1) % 2
            stage = (
                pltpu.async_copy(edge_hbm.at[0, sid, pl.ds((b + 1) * BC, BC)],
                                 rows[ibn], isems[ibn]),
                pltpu.async_copy(edge_hbm.at[1, sid, pl.ds((b + 1) * BC, BC)],
                                 cols[ibn], isems[ibn]),
            )

        def pair_body(p, carry, rv=rv, cv=cv):
            c0 = 2 * p
            c1 = c0 + 1
            pltpu.async_copy(tab.at[cv.at[c1]], gbuf1.at[pl.ds(0, CH)], gsem1)
            pltpu.make_async_copy(tab.at[cv.at[c0]], gbuf0.at[pl.ds(0, CH)], gsem0).wait()
            pltpu.sync_copy(gbuf0.at[pl.ds(0, CH)], acc.at[rv.at[c0]], add=True)

            @pl.when(p != PAIRS - 1)
            def _():
                pltpu.async_copy(tab.at[cv.at[c0 + 2]], gbuf0.at[pl.ds(0, CH)], gsem0)

            pltpu.make_async_copy(tab.at[cv.at[c1]], gbuf1.at[pl.ds(0, CH)], gsem1).wait()
            pltpu.sync_copy(gbuf1.at[pl.ds(0, CH)], acc.at[rv.at[c1]], add=True)
            return carry

        lax.fori_loop(0, PAIRS, pair_body, 0)

        if b + 1 < NBLK:
            stage[0].wait()
            stage[1].wait()
            pltpu.async_copy(tab.at[cols[(b + 1) % 2].at[0]], gbuf0.at[pl.ds(0, CH)], gsem0)

    plsc.subcore_barrier()

    for j in range(ROWS_PER_TILE // SR):
        sl = pl.ds(r0 + j * SR, SR)
        pltpu.sync_copy(acc.at[sl], gbuf0)
        pltpu.sync_copy(gbuf0, out_hbm.at[sl, pl.ds(c0col, HD)])


def _tc_body(x_ref, p_ref, wt_ref, b_ref, g_ref, bt_ref, eps_ref, o_ref):
    xagg = p_ref[:N, :]
    h = eps_ref[0] * x_ref[...] + xagg
    y = jnp.dot(h, wt_ref[...], preferred_element_type=jnp.float32)
    y = y + b_ref[...]
    mu = jnp.mean(y, axis=0, keepdims=True)
    var = jnp.mean((y - mu) * (y - mu), axis=0, keepdims=True)
    yn = g_ref[...] * (y - mu) / jnp.sqrt(var + BN_EPS) + bt_ref[...]
    o_ref[...] = jnp.where(yn > 0.0, yn, jnp.exp(yn) - 1.0)


_tc_layer = pl.pallas_call(
    _tc_body,
    out_shape=jax.ShapeDtypeStruct((N, D), jnp.float32),
    in_specs=[
        pl.BlockSpec(memory_space=pltpu.VMEM),
        pl.BlockSpec(memory_space=pltpu.VMEM),
        pl.BlockSpec(memory_space=pltpu.VMEM),
        pl.BlockSpec(memory_space=pltpu.VMEM),
        pl.BlockSpec(memory_space=pltpu.VMEM),
        pl.BlockSpec(memory_space=pltpu.VMEM),
        pl.BlockSpec(memory_space=pltpu.SMEM),
    ],
    out_specs=pl.BlockSpec(memory_space=pltpu.VMEM),
)


def kernel(x, edge_index, W1, b1, eps1, g1, bt1, W2, b2, eps2, g2, bt2):
    edges = edge_index.reshape(2, NS, NCHUNK, CH)

    p1 = _sc_segment_sum(x, edges)
    h1 = _tc_layer(x, p1, W1.T, b1[None, :], g1[None, :], bt1[None, :],
                   eps1.reshape(1))
    p2 = _sc_segment_sum(h1, edges)
    h2 = _tc_layer(h1, p2, W2.T, b2[None, :], g2[None, :], bt2[None, :],
                   eps2.reshape(1))
    return h2


# speedup vs baseline: 1.0709x; 1.0081x over previous
import functools

import jax
import jax.numpy as jnp
from jax import lax
from jax.experimental import pallas as pl
from jax.experimental.pallas import tpu as pltpu
from jax.experimental.pallas import tpu_sc as plsc

N = 10000
D = 128
E = 320000
BN_EPS = 1e-5

NC = 2
NS = 16
HD = D // NC
CH = 125
NCHUNK = 160
SR = 128
TAB_ROWS = 10240
ROWS_PER_TILE = TAB_ROWS // NS

BC = 40
NBLK = NCHUNK // BC
PAIRS = BC // 2

_mesh = plsc.VectorSubcoreMesh(core_axis_name="c", subcore_axis_name="s")


@functools.partial(
    pl.kernel,
    out_type=jax.ShapeDtypeStruct((TAB_ROWS, D), jnp.float32),
    mesh=_mesh,
    scratch_types=[
        pltpu.VMEM_SHARED((TAB_ROWS, HD), jnp.float32),
        pltpu.VMEM_SHARED((TAB_ROWS, HD), jnp.float32),
        pltpu.VMEM((BC, CH), jnp.int32),
        pltpu.VMEM((BC, CH), jnp.int32),
        pltpu.VMEM((BC, CH), jnp.int32),
        pltpu.VMEM((BC, CH), jnp.int32),
        pltpu.VMEM((SR, HD), jnp.float32),
        pltpu.VMEM((SR, HD), jnp.float32),
        pltpu.SemaphoreType.DMA,
        pltpu.SemaphoreType.DMA,
        pltpu.SemaphoreType.DMA,
        pltpu.SemaphoreType.DMA,
    ],
    compiler_params=pltpu.CompilerParams(use_tc_tiling_on_sc=False),
)
def _sc_segment_sum(x_hbm, edge_hbm, out_hbm,
                    tab, acc, rowA, rowB, colA, colB, gbuf0, gbuf1,
                    gsem0, gsem1, isemA, isemB):
    cid = lax.axis_index("c")
    sid = lax.axis_index("s")
    r0 = sid * ROWS_PER_TILE
    c0col = cid * HD

    for j in range(ROWS_PER_TILE // SR):
        base = r0 + j * SR
        sl = pl.ds(base, SR)

        @pl.when(base + SR <= N)
        def _(sl=sl, base=base):
            pltpu.sync_copy(x_hbm.at[sl, pl.ds(c0col, HD)], gbuf0)
            pltpu.sync_copy(gbuf0, tab.at[sl])
        pltpu.sync_copy(gbuf0, acc.at[sl])

    @pl.when(sid == NS - 1)
    def _():
        tail = N - (N // SR) * SR
        tsl = pl.ds(N - tail, tail)
        pltpu.sync_copy(x_hbm.at[tsl, pl.ds(c0col, HD)],
                        gbuf0.at[pl.ds(0, tail)])
        pltpu.sync_copy(gbuf0.at[pl.ds(0, tail)], tab.at[tsl])
        pltpu.sync_copy(gbuf0.at[pl.ds(0, tail)], acc.at[tsl])

    plsc.subcore_barrier()

    rows = (rowA, rowB)
    cols = (colA, colB)
    isems = (isemA, isemB)

    pltpu.sync_copy(edge_hbm.at[0, sid, pl.ds(0, BC)], rowA)
    pltpu.sync_copy(edge_hbm.at[1, sid, pl.ds(0, BC)], colA)
    pltpu.async_copy(tab.at[colA.at[0]], gbuf0.at[pl.ds(0, CH)], gsem0)

    for b in range(NBLK):
        ib = b % 2
        rv = rows[ib]
        cv = cols[ib]
        stage = None
        if b + 1 < NBLK:
            ibn = (b + 1) % 2
            stage = (
                pltpu.async_copy(edge_hbm.at[0, sid, pl.ds((b + 1) * BC, BC)],
                                 rows[ibn], isems[ibn]),
                pltpu.async_copy(edge_hbm.at[1, sid, pl.ds((b + 1) * BC, BC)],
                                 cols[ibn], isems[ibn]),
            )

        def pair_body(p, carry, rv=rv, cv=cv):
            c0 = 2 * p
            c1 = c0 + 1
            pltpu.async_copy(tab.at[cv.at[c1]], gbuf1.at[pl.ds(0, CH)], gsem1)
            pltpu.make_async_copy(tab.at[cv.at[c0]], gbuf0.at[pl.ds(0, CH)], gsem0).wait()
            pltpu.sync_copy(gbuf0.at[pl.ds(0, CH)], acc.at[rv.at[c0]], add=True)

            @pl.when(p != PAIRS - 1)
            def _():
                pltpu.async_copy(tab.at[cv.at[c0 + 2]], gbuf0.at[pl.ds(0, CH)], gsem0)

            pltpu.make_async_copy(tab.at[cv.at[c1]], gbuf1.at[pl.ds(0, CH)], gsem1).wait()
            pltpu.sync_copy(gbuf1.at[pl.ds(0, CH)], acc.at[rv.at[c1]], add=True)
            return carry

        lax.fori_loop(0, PAIRS, pair_body, 0)

        if b + 1 < NBLK:
            stage[0].wait()
            stage[1].wait()
            pltpu.async_copy(tab.at[cols[(b + 1) % 2].at[0]], gbuf0.at[pl.ds(0, CH)], gsem0)

    plsc.subcore_barrier()

    for j in range(ROWS_PER_TILE // SR):
        sl = pl.ds(r0 + j * SR, SR)
        pltpu.sync_copy(acc.at[sl], gbuf0)
        pltpu.sync_copy(gbuf0, out_hbm.at[sl, pl.ds(c0col, HD)])


def _tc_body(x_ref, p_ref, wt_ref, b_ref, g_ref, bt_ref, eps_ref, o_ref):
    xagg = p_ref[:N, :]
    h = eps_ref[0] * x_ref[...] + xagg
    y = jnp.dot(h, wt_ref[...], preferred_element_type=jnp.float32)
    y = y + b_ref[...]
    mu = jnp.mean(y, axis=0, keepdims=True)
    var = jnp.mean((y - mu) * (y - mu), axis=0, keepdims=True)
    yn = g_ref[...] * (y - mu) / jnp.sqrt(var + BN_EPS) + bt_ref[...]
    o_ref[...] = jnp.where(yn > 0.0, yn, jnp.exp(yn) - 1.0)


_tc_layer = pl.pallas_call(
    _tc_body,
    out_shape=jax.ShapeDtypeStruct((N, D), jnp.float32),
    in_specs=[
        pl.BlockSpec(memory_space=pltpu.VMEM),
        pl.BlockSpec(memory_space=pltpu.VMEM),
        pl.BlockSpec(memory_space=pltpu.VMEM),
        pl.BlockSpec(memory_space=pltpu.VMEM),
        pl.BlockSpec(memory_space=pltpu.VMEM),
        pl.BlockSpec(memory_space=pltpu.VMEM),
        pl.BlockSpec(memory_space=pltpu.SMEM),
    ],
    out_specs=pl.BlockSpec(memory_space=pltpu.VMEM),
)


def kernel(x, edge_index, W1, b1, eps1, g1, bt1, W2, b2, eps2, g2, bt2):
    edges = edge_index.reshape(2, NS, NCHUNK, CH)

    p1 = _sc_segment_sum(x, edges)
    h1 = _tc_layer(x, p1, W1.T, b1[None, :], g1[None, :], bt1[None, :],
                   eps1.reshape(1))
    p2 = _sc_segment_sum(h1, edges)
    h2 = _tc_layer(h1, p2, W2.T, b2[None, :], g2[None, :], bt2[None, :],
                   eps2.reshape(1))
    return h2


# speedup vs baseline: 1.1097x; 1.0362x over previous
import functools

import jax
import jax.numpy as jnp
from jax import lax
from jax.experimental import pallas as pl
from jax.experimental.pallas import tpu as pltpu
from jax.experimental.pallas import tpu_sc as plsc

N = 10000
D = 128
E = 320000
BN_EPS = 1e-5

NC = 2
NS = 16
HD = D // NC
CH = 125
NCHUNK = 160
SR = 128
TAB_ROWS = 10240
ROWS_PER_TILE = TAB_ROWS // NS

BC = 40
NBLK = NCHUNK // BC
PAIRS = BC // 2

_mesh = plsc.VectorSubcoreMesh(core_axis_name="c", subcore_axis_name="s")


@functools.partial(
    pl.kernel,
    out_type=jax.ShapeDtypeStruct((TAB_ROWS, D), jnp.float32),
    mesh=_mesh,
    scratch_types=[
        pltpu.VMEM_SHARED((TAB_ROWS, HD), jnp.float32),
        pltpu.VMEM_SHARED((TAB_ROWS, HD), jnp.float32),
        pltpu.VMEM((BC, CH), jnp.int32),
        pltpu.VMEM((BC, CH), jnp.int32),
        pltpu.VMEM((BC, CH), jnp.int32),
        pltpu.VMEM((BC, CH), jnp.int32),
        pltpu.VMEM((BC, CH), jnp.int32),
        pltpu.VMEM((BC, CH), jnp.int32),
        pltpu.VMEM((SR, HD), jnp.float32),
        pltpu.VMEM((SR, HD), jnp.float32),
        pltpu.SemaphoreType.DMA,
        pltpu.SemaphoreType.DMA,
        pltpu.SemaphoreType.DMA,
        pltpu.SemaphoreType.DMA,
        pltpu.SemaphoreType.DMA,
    ],
    compiler_params=pltpu.CompilerParams(use_tc_tiling_on_sc=False),
)
def _sc_segment_sum(x_hbm, edge_hbm, out_hbm,
                    tab, acc, rowA, rowB, rowC, colA, colB, colC,
                    gbuf0, gbuf1, gsem0, gsem1, isemA, isemB, isemC):
    cid = lax.axis_index("c")
    sid = lax.axis_index("s")
    r0 = sid * ROWS_PER_TILE
    c0col = cid * HD

    for j in range(ROWS_PER_TILE // SR):
        base = r0 + j * SR
        sl = pl.ds(base, SR)

        @pl.when(base + SR <= N)
        def _(sl=sl, base=base):
            pltpu.sync_copy(x_hbm.at[sl, pl.ds(c0col, HD)], gbuf0)
            pltpu.sync_copy(gbuf0, tab.at[sl])
        pltpu.sync_copy(gbuf0, acc.at[sl])

    @pl.when(sid == NS - 1)
    def _():
        tail = N - (N // SR) * SR
        tsl = pl.ds(N - tail, tail)
        pltpu.sync_copy(x_hbm.at[tsl, pl.ds(c0col, HD)],
                        gbuf0.at[pl.ds(0, tail)])
        pltpu.sync_copy(gbuf0.at[pl.ds(0, tail)], tab.at[tsl])
        pltpu.sync_copy(gbuf0.at[pl.ds(0, tail)], acc.at[tsl])

    plsc.subcore_barrier()

    rows = (rowA, rowB, rowC)
    cols = (colA, colB, colC)
    isems = (isemA, isemB, isemC)

    def stage_block(b):
        ib = b % 3
        return (
            pltpu.async_copy(edge_hbm.at[0, sid, pl.ds(b * BC, BC)],
                             rows[ib], isems[ib]),
            pltpu.async_copy(edge_hbm.at[1, sid, pl.ds(b * BC, BC)],
                             cols[ib], isems[ib]),
        )

    for h in stage_block(0):
        h.wait()
    stages = {}
    if NBLK > 1:
        stages[1] = stage_block(1)
    if NBLK > 2:
        stages[2] = stage_block(2)
    pltpu.async_copy(tab.at[colA.at[0]], gbuf0.at[pl.ds(0, CH)], gsem0)

    for b in range(NBLK):
        ib = b % 3
        rv = rows[ib]
        cv = cols[ib]
        if b + 1 < NBLK:
            for h in stages.pop(b + 1):
                h.wait()
        if b + 3 < NBLK:
            stages[b + 3] = stage_block(b + 3)
        cvn = cols[(b + 1) % 3] if b + 1 < NBLK else None

        def pair_body(p, carry, rv=rv, cv=cv, cvn=cvn):
            c0 = 2 * p
            c1 = c0 + 1
            pltpu.async_copy(tab.at[cv.at[c1]], gbuf1.at[pl.ds(0, CH)], gsem1)
            pltpu.make_async_copy(tab.at[cv.at[c0]], gbuf0.at[pl.ds(0, CH)], gsem0).wait()
            pltpu.sync_copy(gbuf0.at[pl.ds(0, CH)], acc.at[rv.at[c0]], add=True)

            @pl.when(p != PAIRS - 1)
            def _():
                pltpu.async_copy(tab.at[cv.at[c0 + 2]], gbuf0.at[pl.ds(0, CH)], gsem0)

            if cvn is not None:
                @pl.when(p == PAIRS - 1)
                def _():
                    pltpu.async_copy(tab.at[cvn.at[0]], gbuf0.at[pl.ds(0, CH)], gsem0)

            pltpu.make_async_copy(tab.at[cv.at[c1]], gbuf1.at[pl.ds(0, CH)], gsem1).wait()
            pltpu.sync_copy(gbuf1.at[pl.ds(0, CH)], acc.at[rv.at[c1]], add=True)
            return carry

        lax.fori_loop(0, PAIRS, pair_body, 0)

    plsc.subcore_barrier()

    for j in range(ROWS_PER_TILE // SR):
        sl = pl.ds(r0 + j * SR, SR)
        pltpu.sync_copy(acc.at[sl], gbuf0)
        pltpu.sync_copy(gbuf0, out_hbm.at[sl, pl.ds(c0col, HD)])


def _tc_body(x_ref, p_ref, wt_ref, b_ref, g_ref, bt_ref, eps_ref, o_ref):
    xagg = p_ref[:N, :]
    h = eps_ref[0] * x_ref[...] + xagg
    y = jnp.dot(h, wt_ref[...], preferred_element_type=jnp.float32)
    y = y + b_ref[...]
    mu = jnp.mean(y, axis=0, keepdims=True)
    var = jnp.mean((y - mu) * (y - mu), axis=0, keepdims=True)
    yn = g_ref[...] * (y - mu) / jnp.sqrt(var + BN_EPS) + bt_ref[...]
    o_ref[...] = jnp.where(yn > 0.0, yn, jnp.exp(yn) - 1.0)


_tc_layer = pl.pallas_call(
    _tc_body,
    out_shape=jax.ShapeDtypeStruct((N, D), jnp.float32),
    in_specs=[
        pl.BlockSpec(memory_space=pltpu.VMEM),
        pl.BlockSpec(memory_space=pltpu.VMEM),
        pl.BlockSpec(memory_space=pltpu.VMEM),
        pl.BlockSpec(memory_space=pltpu.VMEM),
        pl.BlockSpec(memory_space=pltpu.VMEM),
        pl.BlockSpec(memory_space=pltpu.VMEM),
        pl.BlockSpec(memory_space=pltpu.SMEM),
    ],
    out_specs=pl.BlockSpec(memory_space=pltpu.VMEM),
)


def kernel(x, edge_index, W1, b1, eps1, g1, bt1, W2, b2, eps2, g2, bt2):
    edges = edge_index.reshape(2, NS, NCHUNK, CH)

    p1 = _sc_segment_sum(x, edges)
    h1 = _tc_layer(x, p1, W1.T, b1[None, :], g1[None, :], bt1[None, :],
                   eps1.reshape(1))
    p2 = _sc_segment_sum(h1, edges)
    h2 = _tc_layer(h1, p2, W2.T, b2[None, :], g2[None, :], bt2[None, :],
                   eps2.reshape(1))
    return h2
